# Initial kernel scaffold; baseline (speedup 1.0000x reference)
import jax
import jax.numpy as jnp
from jax.experimental import pallas as pl


def kernel(h, edge_index):
    raise NotImplementedError("write your pallas kernel here")


# speedup vs baseline: 1.1766x; 1.1766x over previous
import dataclasses
import functools

import jax
import jax.numpy as jnp
from jax import lax
from jax.experimental import pallas as pl
from jax.experimental.pallas import tpu as pltpu
from jax.experimental.pallas import tpu_sc as plsc

E = 320000
D = 128
NW = 32
EPW = E // NW
C = 80
NCHUNK = EPW // C
L = 16
G = C // L


def _edge_dot_kernel(h_hbm, src_hbm, dst_hbm, out_hbm,
                     src_v, dst_v, u_v, v_v, out_v, sem):
    cid = lax.axis_index("c")
    sid = lax.axis_index("s")
    wid = sid * 2 + cid
    base = wid * EPW

    pltpu.sync_copy(src_hbm.at[pl.ds(base, EPW)], src_v)
    pltpu.sync_copy(dst_hbm.at[pl.ds(base, EPW)], dst_v)

    lane = lax.iota(jnp.int32, L)

    @pl.loop(0, NCHUNK)
    def _chunk(ci):
        off = ci * C
        cp_u = pltpu.async_copy(h_hbm.at[src_v.at[pl.ds(off, C)]], u_v, sem)
        cp_v = pltpu.async_copy(h_hbm.at[dst_v.at[pl.ds(off, C)]], v_v, sem)
        cp_u.wait()
        cp_v.wait()

        for g in range(G):
            e16 = lane + (g * L)

            def body(k, acc):
                kk = lax.broadcast(k, (L,))
                u = plsc.load_gather(u_v, [e16, kk])
                v = plsc.load_gather(v_v, [e16, kk])
                return acc + u * v

            acc = lax.fori_loop(0, D, body, jnp.zeros((L,), jnp.float32))
            out_v[pl.ds(off + g * L, L)] = acc

    pltpu.sync_copy(out_v, out_hbm.at[pl.ds(base, EPW)])


@jax.jit
def kernel(h, edge_index):
    edge_index = edge_index.astype(jnp.int32)
    src = edge_index[0]
    dst = edge_index[1]

    mesh = plsc.VectorSubcoreMesh(core_axis_name="c", subcore_axis_name="s")
    cp = pltpu.CompilerParams()
    if "needs_layout_passes" in pltpu.CompilerParams.__dataclass_fields__:
        cp = dataclasses.replace(cp, needs_layout_passes=False)
    k = pl.kernel(
        _edge_dot_kernel,
        out_type=jax.ShapeDtypeStruct((E,), jnp.float32),
        mesh=mesh,
        scratch_types=[
            pltpu.VMEM((EPW,), jnp.int32),
            pltpu.VMEM((EPW,), jnp.int32),
            pltpu.VMEM((C, D), jnp.float32),
            pltpu.VMEM((C, D), jnp.float32),
            pltpu.VMEM((EPW,), jnp.float32),
            pltpu.SemaphoreType.DMA,
        ],
        compiler_params=cp,
    )
    score = k(h, src, dst)
    return score.reshape(E, 1)
